# baseline (speedup 1.0000x reference)
import functools

import jax
import jax.numpy as jnp
from jax import lax
from jax.experimental import pallas as pl
from jax.experimental.pallas import tpu as pltpu
from jax.experimental.pallas import tpu_sc as plsc


@functools.lru_cache(maxsize=None)
def _make_gather(B: int, D: int):
    info = plsc.get_sparse_core_info()
    nc, ns = info.num_cores, info.num_subcores
    nw = nc * ns
    bpw = B // nw
    mesh = plsc.VectorSubcoreMesh(core_axis_name="c", subcore_axis_name="s")

    @functools.partial(
        pl.kernel,
        mesh=mesh,
        compiler_params=pltpu.CompilerParams(use_tc_tiling_on_sc=False),
        out_type=[
            jax.ShapeDtypeStruct((B, D), jnp.float32),
            jax.ShapeDtypeStruct((B, D), jnp.float32),
        ],
        scratch_types=[
            pltpu.VMEM((bpw,), jnp.int32),
            pltpu.VMEM((bpw,), jnp.int32),
            pltpu.VMEM((bpw, D), jnp.float32),
            pltpu.VMEM((bpw, D), jnp.float32),
            pltpu.SemaphoreType.DMA,
            pltpu.SemaphoreType.DMA,
        ],
    )
    def gather(users_hbm, items_hbm, utab_hbm, itab_hbm, ue_hbm, ie_hbm,
               uidx, iidx, urows, irows, usem, isem):
        wid = lax.axis_index("s") * nc + lax.axis_index("c")
        base = wid * bpw
        pltpu.sync_copy(users_hbm.at[pl.ds(base, bpw)], uidx)
        pltpu.sync_copy(items_hbm.at[pl.ds(base, bpw)], iidx)
        cu = pltpu.async_copy(utab_hbm.at[uidx], urows, usem)
        ci = pltpu.async_copy(itab_hbm.at[iidx], irows, isem)
        cu.wait()
        pltpu.sync_copy(urows, ue_hbm.at[pl.ds(base, bpw)])
        ci.wait()
        pltpu.sync_copy(irows, ie_hbm.at[pl.ds(base, bpw)])

    return gather


def _mlp_body(ue_ref, ie_ref, w1u_ref, w1i_ref, b1_ref, w2_ref, b2_ref,
              w3_ref, b3_ref, out_ref):
    h = jnp.dot(ue_ref[...], w1u_ref[...], preferred_element_type=jnp.float32)
    h = h + jnp.dot(ie_ref[...], w1i_ref[...], preferred_element_type=jnp.float32)
    h = jnp.maximum(h + b1_ref[...], 0.0)
    h = jnp.dot(h, w2_ref[...], preferred_element_type=jnp.float32)
    h = jnp.maximum(h + b2_ref[...], 0.0)
    o = jnp.dot(h, w3_ref[...], preferred_element_type=jnp.float32)
    out_ref[...] = jax.nn.sigmoid(o + b3_ref[...])


@functools.lru_cache(maxsize=None)
def _make_mlp(B: int, D: int, H1: int, H2: int, BM: int):
    rep = lambda i: (0, 0)
    return pl.pallas_call(
        _mlp_body,
        grid=(B // BM,),
        in_specs=[
            pl.BlockSpec((BM, D), lambda i: (i, 0)),
            pl.BlockSpec((BM, D), lambda i: (i, 0)),
            pl.BlockSpec((D, H1), rep),
            pl.BlockSpec((D, H1), rep),
            pl.BlockSpec((1, H1), rep),
            pl.BlockSpec((H1, H2), rep),
            pl.BlockSpec((1, H2), rep),
            pl.BlockSpec((H2, 1), rep),
            pl.BlockSpec((1, 1), rep),
        ],
        out_specs=pl.BlockSpec((BM, 1), lambda i: (i, 0)),
        out_shape=jax.ShapeDtypeStruct((B, 1), jnp.float32),
    )


def kernel(users, items, user_emb_w, item_emb_w, W1, b1, W2, b2, W3, b3):
    B = users.shape[0]
    D = user_emb_w.shape[1]
    H1 = W1.shape[0]
    H2 = W2.shape[0]

    ue, ie = _make_gather(B, D)(users.astype(jnp.int32), items.astype(jnp.int32),
                                user_emb_w, item_emb_w)

    w1u = W1[:, :D].T
    w1i = W1[:, D:].T
    out = _make_mlp(B, D, H1, H2, 2048)(
        ue, ie, w1u, w1i, b1.reshape(1, H1), W2.T, b2.reshape(1, H2),
        W3.T, b3.reshape(1, 1))
    return out[:, 0]
